# baseline (speedup 1.0000x reference)
import functools

import jax
import jax.numpy as jnp
from jax import lax
from jax.experimental import pallas as pl
from jax.experimental.pallas import tpu as pltpu
from jax.experimental.pallas import tpu_sc as plsc

B = 512
D = 128
N_ATOMS = 10000
N_BOND_ROWS = 320000
N_BONDS = N_BOND_ROWS // 2

NC = 2
NS = 16
NW = NC * NS

BHALF = 128
BOND_PT = 5120
NB_CHUNKS = BOND_PT // BHALF
BOND_LAST = N_BONDS - BHALF

DUMMY = B
ACC_ROWS = 520

_mesh = plsc.VectorSubcoreMesh(core_axis_name="c", subcore_axis_name="s")


@functools.partial(
    pl.kernel,
    out_type=jax.ShapeDtypeStruct((NC, B, D), jnp.float32),
    mesh=_mesh,
    scratch_types=[
        pltpu.VMEM((NB_CHUNKS, BHALF), jnp.int32),
        pltpu.VMEM((BHALF, 2, D), jnp.float32),
        pltpu.VMEM((BHALF, 2, D), jnp.float32),
        pltpu.VMEM((BHALF, 2, D), jnp.float32),
        pltpu.VMEM((32, D), jnp.float32),
        pltpu.VMEM((32, 2, D), jnp.float32),
        pltpu.VMEM_SHARED((ACC_ROWS, 2, D), jnp.float32),
        pltpu.SemaphoreType.DMA,
        pltpu.SemaphoreType.DMA,
        pltpu.SemaphoreType.DMA,
    ],
)
def _sc_bond_pool(bond_hbm, bid_hbm, pb_hbm,
                  bids_v, buf0, buf1, buf2, ebuf, zbufb, acc_b, semA, semB, semC):
    cid = lax.axis_index("c")
    sid = lax.axis_index("s")
    wid = cid * NS + sid

    zvec = jnp.zeros((16,), jnp.float32)
    for r in range(32):
        for h in range(2):
            for g in range(D // 16):
                zbufb[r, h, pl.ds(g * 16, 16)] = zvec
    pltpu.sync_copy(zbufb, acc_b.at[pl.ds(sid * 32, 32)])

    @pl.when(sid == 0)
    def _zero_dummy_rows():
        pltpu.sync_copy(zbufb.at[pl.ds(0, 8)], acc_b.at[pl.ds(B, 8)])

    pltpu.sync_copy(bid_hbm.at[wid], bids_v)

    def bstart(j):
        return pl.multiple_of(jnp.minimum(wid * BOND_PT + j * BHALF, BOND_LAST), 8)

    plsc.subcore_barrier()

    pltpu.async_copy(bond_hbm.at[pl.ds(bstart(0), BHALF)], buf0, semA)
    pltpu.async_copy(bond_hbm.at[pl.ds(bstart(1), BHALF)], buf1, semB)
    pltpu.async_copy(bond_hbm.at[pl.ds(bstart(2), BHALF)], buf2, semC)

    def bond_triple(p, carry):
        j = 3 * p
        for off, (buf, sem) in enumerate(((buf0, semA), (buf1, semB),
                                          (buf2, semC))):
            pltpu.make_async_copy(bond_hbm.at[pl.ds(bstart(j + off), BHALF)],
                                  buf, sem).wait()
            pltpu.sync_copy(buf, acc_b.at[bids_v.at[j + off]], add=True)
            pltpu.async_copy(bond_hbm.at[pl.ds(bstart(j + off + 3), BHALF)],
                             buf, sem)
        return carry

    lax.fori_loop(0, (NB_CHUNKS - 4) // 3, bond_triple, 0)
    j = NB_CHUNKS - 4
    pltpu.make_async_copy(bond_hbm.at[pl.ds(bstart(j), BHALF)], buf0, semA).wait()
    pltpu.sync_copy(buf0, acc_b.at[bids_v.at[j]], add=True)
    pltpu.async_copy(bond_hbm.at[pl.ds(bstart(j + 3), BHALF)], buf0, semA)
    pltpu.make_async_copy(bond_hbm.at[pl.ds(bstart(j + 1), BHALF)], buf1, semB).wait()
    pltpu.sync_copy(buf1, acc_b.at[bids_v.at[j + 1]], add=True)
    pltpu.make_async_copy(bond_hbm.at[pl.ds(bstart(j + 2), BHALF)], buf2, semC).wait()
    pltpu.sync_copy(buf2, acc_b.at[bids_v.at[j + 2]], add=True)
    pltpu.make_async_copy(bond_hbm.at[pl.ds(bstart(j + 3), BHALF)], buf0, semA).wait()
    pltpu.sync_copy(buf0, acc_b.at[bids_v.at[j + 3]], add=True)

    plsc.subcore_barrier()

    pltpu.sync_copy(acc_b.at[pl.ds(sid * 32, 32)], zbufb)
    for r in range(32):
        for g in range(D // 16):
            ebuf[r, pl.ds(g * 16, 16)] = zbufb[r, 0, pl.ds(g * 16, 16)]
    pltpu.sync_copy(ebuf, pb_hbm.at[cid, pl.ds(sid * 32, 32)])


def _combine_body(atom_ref, aid_ref, pb_ref, g_ref, o_ref):
    seg = lax.broadcasted_iota(jnp.int32, (B, N_ATOMS), 0)
    one_hot = jnp.where(aid_ref[:] == seg, 1.0, 0.0).astype(jnp.bfloat16)
    hi = atom_ref[:].astype(jnp.bfloat16)
    lo = (atom_ref[:] - hi.astype(jnp.float32)).astype(jnp.bfloat16)
    o_ref[:, 0:D] = (
        jax.lax.dot(one_hot, hi, preferred_element_type=jnp.float32)
        + jax.lax.dot(one_hot, lo, preferred_element_type=jnp.float32))
    o_ref[:, D:2 * D] = pb_ref[0] + pb_ref[1]
    o_ref[:, 2 * D:3 * D] = g_ref[:]


_combine = pl.pallas_call(
    _combine_body,
    out_shape=jax.ShapeDtypeStruct((B, 3 * D), jnp.float32),
)


def _dummy_chunks(n_chunks, width):
    v = DUMMY + (jnp.arange(n_chunks, dtype=jnp.int32) % 8)
    return jnp.repeat(v, width)


def _bond_ids_laid(bid):
    total = NW * NB_CHUNKS * BHALF
    n_tail = (total - N_BONDS) // BHALF
    laid = jnp.concatenate([bid, _dummy_chunks(n_tail, BHALF)])
    return laid.reshape(NW, NB_CHUNKS, BHALF)


def kernel(atom_feats, bond_feats, global_feats, atom_segment_ids, bond_segment_ids):
    aid = atom_segment_ids.astype(jnp.int32)
    bid = bond_segment_ids.astype(jnp.int32)
    bid_p = _bond_ids_laid(bid)
    bond3 = bond_feats.reshape(N_BONDS, 2, D)
    pb = _sc_bond_pool(bond3, bid_p)
    return _combine(atom_feats, aid.reshape(1, N_ATOMS), pb, global_feats)


# speedup vs baseline: 1.0279x; 1.0279x over previous
import functools

import jax
import jax.numpy as jnp
from jax import lax
from jax.experimental import pallas as pl
from jax.experimental.pallas import tpu as pltpu
from jax.experimental.pallas import tpu_sc as plsc

B = 512
D = 128
N_ATOMS = 10000
N_BOND_ROWS = 320000
N_BONDS = N_BOND_ROWS // 2

NC = 2
NS = 16
NW = NC * NS

BHALF = 128
BOND_PT = 5120
NB_CHUNKS = BOND_PT // BHALF
BOND_LAST = N_BONDS - BHALF

DUMMY = B
ACC_ROWS = 520

_mesh = plsc.VectorSubcoreMesh(core_axis_name="c", subcore_axis_name="s")


@functools.partial(
    pl.kernel,
    out_type=jax.ShapeDtypeStruct((NC, B, D), jnp.float32),
    mesh=_mesh,
    scratch_types=[
        pltpu.VMEM((NB_CHUNKS, BHALF), jnp.int32),
        pltpu.VMEM((BHALF, 2, D), jnp.float32),
        pltpu.VMEM((BHALF, 2, D), jnp.float32),
        pltpu.VMEM((BHALF, 2, D), jnp.float32),
        pltpu.VMEM((32, D), jnp.float32),
        pltpu.VMEM((32, 2, D), jnp.float32),
        pltpu.VMEM_SHARED((ACC_ROWS, 2, D), jnp.float32),
        pltpu.SemaphoreType.DMA,
        pltpu.SemaphoreType.DMA,
        pltpu.SemaphoreType.DMA,
    ],
)
def _sc_bond_pool(bond_hbm, bid_hbm, pb_hbm,
                  bids_v, buf0, buf1, buf2, ebuf, zbufb, acc_b, semA, semB, semC):
    cid = lax.axis_index("c")
    sid = lax.axis_index("s")
    wid = cid * NS + sid

    zvec = jnp.zeros((16,), jnp.float32)
    for r in range(32):
        for h in range(2):
            for g in range(D // 16):
                zbufb[r, h, pl.ds(g * 16, 16)] = zvec
    pltpu.sync_copy(zbufb, acc_b.at[pl.ds(sid * 32, 32)])

    @pl.when(sid == 0)
    def _zero_dummy_rows():
        pltpu.sync_copy(zbufb.at[pl.ds(0, 8)], acc_b.at[pl.ds(B, 8)])

    pltpu.sync_copy(bid_hbm.at[wid], bids_v)

    def bstart(j):
        return pl.multiple_of(jnp.minimum(wid * BOND_PT + j * BHALF, BOND_LAST), 8)

    plsc.subcore_barrier()

    pltpu.async_copy(bond_hbm.at[pl.ds(bstart(0), BHALF)], buf0, semA)
    pltpu.async_copy(bond_hbm.at[pl.ds(bstart(1), BHALF)], buf1, semB)
    pltpu.async_copy(bond_hbm.at[pl.ds(bstart(2), BHALF)], buf2, semC)

    def bond_triple(p, carry):
        j = 3 * p
        for off, (buf, sem) in enumerate(((buf0, semA), (buf1, semB),
                                          (buf2, semC))):
            pltpu.make_async_copy(bond_hbm.at[pl.ds(bstart(j + off), BHALF)],
                                  buf, sem).wait()
            pltpu.sync_copy(buf, acc_b.at[bids_v.at[j + off]], add=True)
            pltpu.async_copy(bond_hbm.at[pl.ds(bstart(j + off + 3), BHALF)],
                             buf, sem)
        return carry

    lax.fori_loop(0, (NB_CHUNKS - 4) // 3, bond_triple, 0)
    j = NB_CHUNKS - 4
    pltpu.make_async_copy(bond_hbm.at[pl.ds(bstart(j), BHALF)], buf0, semA).wait()
    pltpu.sync_copy(buf0, acc_b.at[bids_v.at[j]], add=True)
    pltpu.async_copy(bond_hbm.at[pl.ds(bstart(j + 3), BHALF)], buf0, semA)
    pltpu.make_async_copy(bond_hbm.at[pl.ds(bstart(j + 1), BHALF)], buf1, semB).wait()
    pltpu.sync_copy(buf1, acc_b.at[bids_v.at[j + 1]], add=True)
    pltpu.make_async_copy(bond_hbm.at[pl.ds(bstart(j + 2), BHALF)], buf2, semC).wait()
    pltpu.sync_copy(buf2, acc_b.at[bids_v.at[j + 2]], add=True)
    pltpu.make_async_copy(bond_hbm.at[pl.ds(bstart(j + 3), BHALF)], buf0, semA).wait()
    pltpu.sync_copy(buf0, acc_b.at[bids_v.at[j + 3]], add=True)

    plsc.subcore_barrier()

    pltpu.sync_copy(acc_b.at[pl.ds(sid * 32, 32)], zbufb)
    for r in range(32):
        for g in range(D // 16):
            ebuf[r, pl.ds(g * 16, 16)] = zbufb[r, 0, pl.ds(g * 16, 16)]
    pltpu.sync_copy(ebuf, pb_hbm.at[cid, pl.ds(sid * 32, 32)])


def _combine_body(atom_ref, aid_ref, pb_ref, g_ref, o_ref):
    seg = lax.broadcasted_iota(jnp.int32, (B, N_ATOMS), 0)
    one_hot = jnp.where(aid_ref[:] == seg, 1.0, 0.0).astype(jnp.float32)
    o_ref[:, 0:D] = jax.lax.dot(one_hot, atom_ref[:],
                                preferred_element_type=jnp.float32)
    o_ref[:, D:2 * D] = pb_ref[0] + pb_ref[1]
    o_ref[:, 2 * D:3 * D] = g_ref[:]


_combine = pl.pallas_call(
    _combine_body,
    out_shape=jax.ShapeDtypeStruct((B, 3 * D), jnp.float32),
)


def _dummy_chunks(n_chunks, width):
    v = DUMMY + (jnp.arange(n_chunks, dtype=jnp.int32) % 8)
    return jnp.repeat(v, width)


def _bond_ids_laid(bid):
    total = NW * NB_CHUNKS * BHALF
    n_tail = (total - N_BONDS) // BHALF
    laid = jnp.concatenate([bid, _dummy_chunks(n_tail, BHALF)])
    return laid.reshape(NW, NB_CHUNKS, BHALF)


def kernel(atom_feats, bond_feats, global_feats, atom_segment_ids, bond_segment_ids):
    aid = atom_segment_ids.astype(jnp.int32)
    bid = bond_segment_ids.astype(jnp.int32)
    bid_p = _bond_ids_laid(bid)
    bond3 = bond_feats.reshape(N_BONDS, 2, D)
    pb = _sc_bond_pool(bond3, bid_p)
    return _combine(atom_feats, aid.reshape(1, N_ATOMS), pb, global_feats)
